# baseline (speedup 1.0000x reference)
import jax
import jax.numpy as jnp
from jax.experimental import pallas as pl

_B, _D = 16384, 64
_BLK = 16384
_EPS = 1e-12


def _body(gu_ref, gi_ref, gis_ref, xui_ref, guo_ref, gio_ref, giso_ref):
    gu = gu_ref[...]
    gi = gi_ref[...]
    gis = gis_ref[...]
    guo_ref[...] = gu
    gio_ref[...] = gi
    giso_ref[...] = gis
    c = jnp.sum(gis * gis, axis=0)
    inv = 1.0 / jnp.maximum(jnp.sqrt(c), _EPS)
    f = gi + gis * inv[None, :]
    xui_ref[...] = jnp.sum(gu * f, axis=0)


def kernel(gu, gi, gis):
    guT = gu.T
    giT = gi.T
    gisT = gis.T
    col = pl.BlockSpec((_D, _BLK), lambda i: (0, i))
    xui, guoT, gioT, gisoT = pl.pallas_call(
        _body,
        grid=(_B // _BLK,),
        in_specs=[col, col, col],
        out_specs=(pl.BlockSpec((_BLK,), lambda i: (i,)), col, col, col),
        out_shape=(
            jax.ShapeDtypeStruct((_B,), jnp.float32),
            jax.ShapeDtypeStruct((_D, _B), jnp.float32),
            jax.ShapeDtypeStruct((_D, _B), jnp.float32),
            jax.ShapeDtypeStruct((_D, _B), jnp.float32),
        ),
    )(guT, giT, gisT)
    return (xui, guoT.T, gioT.T, gisoT.T)
